# baseline (speedup 1.0000x reference)
import functools

import jax
import jax.numpy as jnp
from jax import lax
from jax.experimental import pallas as pl
from jax.experimental.pallas import tpu as pltpu
from jax.experimental.pallas import tpu_sc as plsc

_NUM_NODE_TYPES = 4
_NUM_EDGE_TYPES = 6
_D = 128
_N = 50000
_E = 800000
_ROWS = _N + _E

_CHUNK = 400
_SUB = 80
_NSUB = _CHUNK // _SUB
_NCHUNKS = _ROWS // _CHUNK


def _make_sc_kernel(nworkers: int):
    mesh = plsc.VectorSubcoreMesh(core_axis_name="c", subcore_axis_name="s")
    max_k = -(-_NCHUNKS // nworkers)
    n_outer = (max_k + 1) // 2

    @functools.partial(
        pl.kernel,
        mesh=mesh,
        out_type=jax.ShapeDtypeStruct((_ROWS, _D), jnp.float32),
        scratch_types=[
            pltpu.VMEM((_NSUB, _SUB), jnp.int32),
            pltpu.VMEM((_NSUB, _SUB), jnp.int32),
            pltpu.VMEM((_CHUNK, _D), jnp.float32),
            pltpu.VMEM((_CHUNK, _D), jnp.float32),
            pltpu.SemaphoreType.DMA,
            pltpu.SemaphoreType.DMA,
            pltpu.SemaphoreType.DMA,
            pltpu.SemaphoreType.DMA,
        ],
    )
    def k(ids_hbm, table_hbm, out_hbm,
          idx0, idx1, rows0, rows1, gsem0, gsem1, ssem0, ssem1):
        info = plsc.get_sparse_core_info()
        ns = info.num_subcores
        wid = lax.axis_index("c") * ns + lax.axis_index("s")
        idxs = (idx0, idx1)
        rowss = (rows0, rows1)
        gsems = (gsem0, gsem1)
        ssems = (ssem0, ssem1)

        def step(k2, carry):
            for b in range(2):
                kk = k2 * 2 + b
                ci = wid + nworkers * kk
                valid = ci < _NCHUNKS
                idx = idxs[b]
                rows = rowss[b]

                @pl.when(jnp.logical_and(valid, kk >= 2))
                def _():
                    pltpu.make_async_copy(
                        rows, out_hbm.at[pl.ds(0, _CHUNK)], ssems[b]).wait()

                @pl.when(valid)
                def _():
                    pltpu.sync_copy(ids_hbm.at[ci], idx)
                    descs = [
                        pltpu.async_copy(
                            table_hbm.at[idx.at[j]],
                            rows.at[pl.ds(j * _SUB, _SUB)],
                            gsems[b])
                        for j in range(_NSUB)
                    ]
                    for d in descs:
                        d.wait()
                    pltpu.async_copy(
                        rows, out_hbm.at[pl.ds(ci * _CHUNK, _CHUNK)], ssems[b])
            return carry

        lax.fori_loop(0, n_outer, step, 0)
        for b in range(2):
            pltpu.make_async_copy(
                rowss[b], out_hbm.at[pl.ds(0, _CHUNK)], ssems[b]).wait()

    return k


_sc_kernel = _make_sc_kernel(32)


def kernel(node_type_ids, edge_type_ids, node_type_table, edge_type_table):
    table = jnp.concatenate([node_type_table, edge_type_table], axis=0)
    ids = jnp.concatenate(
        [node_type_ids.astype(jnp.int32),
         edge_type_ids.astype(jnp.int32) + _NUM_NODE_TYPES])
    ids = ids.reshape(_NCHUNKS, _NSUB, _SUB)
    return _sc_kernel(ids, table)
